# baseline (speedup 1.0000x reference)
import functools

import jax
import jax.numpy as jnp
from jax import lax
from jax.experimental import pallas as pl
from jax.experimental.pallas import tpu as pltpu
from jax.experimental.pallas import tpu_sc as plsc

VOCAB = 100000
EMB = 64
BATCH = 1024
CTX = 20

NUM_CORES = 2
NUM_SUBCORES = 16
NUM_WORKERS = NUM_CORES * NUM_SUBCORES
BPW = BATCH // NUM_WORKERS
IPW = BPW * CTX

LANES = 16
VB = 2048


PLANES_PER_WORKER = EMB // NUM_WORKERS


def _pooled_sc(idx_t, emb_t):
    mesh = plsc.VectorSubcoreMesh(core_axis_name="c", subcore_axis_name="s")

    @functools.partial(
        pl.kernel,
        mesh=mesh,
        out_type=jax.ShapeDtypeStruct((EMB, BATCH), jnp.float32),
        scratch_types=[
            pltpu.VMEM((VOCAB,), jnp.float32),
            pltpu.VMEM((CTX, BATCH), jnp.int32),
            pltpu.VMEM((PLANES_PER_WORKER, BATCH), jnp.float32),
            pltpu.SemaphoreType.DMA,
        ],
        compiler_params=pltpu.CompilerParams(
            use_tc_tiling_on_sc=True, needs_layout_passes=False
        ),
    )
    def k(emb_hbm, idx_hbm, out_hbm, plane_v, idx_v, pool_v, sem):
        wid = lax.axis_index("s") * NUM_CORES + lax.axis_index("c")
        pltpu.sync_copy(idx_hbm, idx_v)
        for r in range(PLANES_PER_WORKER):
            kplane = wid * PLANES_PER_WORKER + r
            pltpu.sync_copy(emb_hbm.at[kplane], plane_v)

            def group_body(g, carry, r=r):
                acc = jnp.zeros((LANES,), jnp.float32)
                for c in range(CTX):
                    idxs = idx_v[c, pl.ds(g * LANES, LANES)]
                    acc = acc + plsc.load_gather(plane_v, [idxs])
                pool_v[r, pl.ds(g * LANES, LANES)] = acc
                return carry

            lax.fori_loop(0, BATCH // LANES, group_body, 0)
        pltpu.sync_copy(
            pool_v, out_hbm.at[pl.ds(wid * PLANES_PER_WORKER, PLANES_PER_WORKER)]
        )

    return k(emb_t, idx_t)


NSTEPS = 49
TAIL = VOCAB - (NSTEPS - 1) * VB
NBUF = 6


def _mm_body(p_ref, wt_ref, o_hbm, acc, sems):
    i = pl.program_id(0)
    slot = lax.rem(i, NBUF)

    for k in range(NBUF):
        @pl.when(jnp.logical_and(i >= NBUF, slot == k))
        def _(k=k):
            pltpu.make_async_copy(
                acc.at[k],
                o_hbm.at[pl.ds((i - NBUF) * VB, VB)],
                sems.at[k],
            ).wait()

    acc[slot] = lax.dot_general(
        wt_ref[...],
        p_ref[...],
        dimension_numbers=(((0,), (0,)), ((), ())),
        preferred_element_type=jnp.float32,
    )

    for k in range(NBUF):
        @pl.when(jnp.logical_and(i < NSTEPS - 1, slot == k))
        def _(k=k):
            pltpu.make_async_copy(
                acc.at[k], o_hbm.at[pl.ds(i * VB, VB)], sems.at[k]
            ).start()

    @pl.when(i == NSTEPS - 1)
    def _():
        last = NSTEPS - 1
        pltpu.make_async_copy(
            acc.at[last % NBUF, pl.ds(0, TAIL)],
            o_hbm.at[pl.ds(last * VB, TAIL)],
            sems.at[last % NBUF],
        ).start()
        for j in range(NSTEPS - NBUF, NSTEPS - 1):
            pltpu.make_async_copy(
                acc.at[j % NBUF],
                o_hbm.at[pl.ds(j * VB, VB)],
                sems.at[j % NBUF],
            ).wait()
        pltpu.make_async_copy(
            acc.at[last % NBUF, pl.ds(0, TAIL)],
            o_hbm.at[pl.ds(last * VB, TAIL)],
            sems.at[last % NBUF],
        ).wait()


def kernel(x, emb_table, W):
    idx_t = x.T.astype(jnp.int32)
    emb_t = emb_table.T
    pooled_t = _pooled_sc(idx_t, emb_t)
    wt = W.T
    out_t = pl.pallas_call(
        _mm_body,
        grid=(NSTEPS,),
        in_specs=[
            pl.BlockSpec((EMB, BATCH), lambda i: (0, 0)),
            pl.BlockSpec((EMB, VB), lambda i: (0, i)),
        ],
        out_specs=pl.BlockSpec(memory_space=pl.ANY),
        out_shape=jax.ShapeDtypeStruct((VOCAB, BATCH), jnp.float32),
        scratch_shapes=[
            pltpu.VMEM((NBUF, VB, BATCH), jnp.float32),
            pltpu.SemaphoreType.DMA((NBUF,)),
        ],
        compiler_params=pltpu.CompilerParams(
            dimension_semantics=("arbitrary",),
        ),
    )(pooled_t, wt)
    return out_t.T


# speedup vs baseline: 1.0067x; 1.0067x over previous
import functools

import jax
import jax.numpy as jnp
from jax import lax
from jax.experimental import pallas as pl
from jax.experimental.pallas import tpu as pltpu
from jax.experimental.pallas import tpu_sc as plsc

VOCAB = 100000
EMB = 64
BATCH = 1024
CTX = 20

NUM_CORES = 2
NUM_SUBCORES = 16
NUM_WORKERS = NUM_CORES * NUM_SUBCORES
BPW = BATCH // NUM_WORKERS
IPW = BPW * CTX

LANES = 16
VB = 2048


PLANES_PER_WORKER = EMB // NUM_WORKERS


def _pooled_sc(idx_t, emb_t):
    mesh = plsc.VectorSubcoreMesh(core_axis_name="c", subcore_axis_name="s")

    @functools.partial(
        pl.kernel,
        mesh=mesh,
        out_type=jax.ShapeDtypeStruct((EMB, BATCH), jnp.float32),
        scratch_types=[
            pltpu.VMEM((50080,), jnp.float32),
            pltpu.VMEM((50080,), jnp.float32),
            pltpu.VMEM((CTX, BATCH), jnp.int32),
            pltpu.VMEM((PLANES_PER_WORKER, BATCH), jnp.float32),
            pltpu.SemaphoreType.DMA((2,)),
        ],
        compiler_params=pltpu.CompilerParams(
            use_tc_tiling_on_sc=True, needs_layout_passes=False
        ),
    )
    def k(emb_hbm, idx_hbm, out_hbm, half_a, half_b, idx_v, pool_v, sems):
        wid = lax.axis_index("s") * NUM_CORES + lax.axis_index("c")
        halves = ((0, 49920), (49920, VOCAB - 49920))
        nchunks = PLANES_PER_WORKER * 2

        bufs = (half_a, half_b)

        def chunk_copy(ci, buf):
            r, h = divmod(ci, 2)
            off, sz = halves[h]
            kplane = wid * PLANES_PER_WORKER + r
            return pltpu.make_async_copy(
                emb_hbm.at[kplane].at[pl.ds(off, sz)],
                bufs[buf].at[pl.ds(0, sz)],
                sems.at[buf],
            )

        chunk_copy(0, 0).start()
        pltpu.sync_copy(idx_hbm, idx_v)
        for ci in range(nchunks):
            buf = ci % 2
            chunk_copy(ci, buf).wait()
            if ci + 1 < nchunks:
                chunk_copy(ci + 1, 1 - buf).start()
            r, h = divmod(ci, 2)
            off, sz = halves[h]
            lo = jnp.full((LANES,), off, jnp.int32)
            hi = jnp.full((LANES,), sz, jnp.int32)

            def group_body(g, carry, buf=buf, h=h, r=r, lo=lo, hi=hi):
                acc = jnp.zeros((LANES,), jnp.float32)
                for c in range(CTX):
                    idxs = idx_v[c, pl.ds(g * LANES, LANES)]
                    loc = idxs - lo
                    msk = jnp.logical_and(loc >= 0, loc < hi)
                    vals = plsc.load_gather(bufs[buf], [loc], mask=msk)
                    acc = acc + jnp.where(msk, vals, 0.0)
                if h == 0:
                    pool_v[r, pl.ds(g * LANES, LANES)] = acc
                else:
                    prev = pool_v[r, pl.ds(g * LANES, LANES)]
                    pool_v[r, pl.ds(g * LANES, LANES)] = prev + acc
                return carry

            lax.fori_loop(0, BATCH // LANES, group_body, 0)
        pltpu.sync_copy(
            pool_v, out_hbm.at[pl.ds(wid * PLANES_PER_WORKER, PLANES_PER_WORKER)]
        )

    return k(emb_t, idx_t)


NSTEPS = 49
TAIL = VOCAB - (NSTEPS - 1) * VB
NBUF = 4


def _mm_body(p_ref, wt_ref, o_hbm, acc, sems):
    i = pl.program_id(0)
    slot = lax.rem(i, NBUF)

    for k in range(NBUF):
        @pl.when(jnp.logical_and(i >= NBUF, slot == k))
        def _(k=k):
            pltpu.make_async_copy(
                acc.at[k],
                o_hbm.at[pl.ds((i - NBUF) * VB, VB)],
                sems.at[k],
            ).wait()

    acc[slot] = lax.dot_general(
        wt_ref[...],
        p_ref[...],
        dimension_numbers=(((0,), (0,)), ((), ())),
        preferred_element_type=jnp.float32,
    )

    for k in range(NBUF):
        @pl.when(jnp.logical_and(i < NSTEPS - 1, slot == k))
        def _(k=k):
            pltpu.make_async_copy(
                acc.at[k], o_hbm.at[pl.ds(i * VB, VB)], sems.at[k]
            ).start()

    @pl.when(i == NSTEPS - 1)
    def _():
        last = NSTEPS - 1
        pltpu.make_async_copy(
            acc.at[last % NBUF, pl.ds(0, TAIL)],
            o_hbm.at[pl.ds(last * VB, TAIL)],
            sems.at[last % NBUF],
        ).start()
        for j in range(NSTEPS - NBUF, NSTEPS - 1):
            pltpu.make_async_copy(
                acc.at[j % NBUF],
                o_hbm.at[pl.ds(j * VB, VB)],
                sems.at[j % NBUF],
            ).wait()
        pltpu.make_async_copy(
            acc.at[last % NBUF, pl.ds(0, TAIL)],
            o_hbm.at[pl.ds(last * VB, TAIL)],
            sems.at[last % NBUF],
        ).wait()


def kernel(x, emb_table, W):
    idx_t = x.T.astype(jnp.int32)
    emb_t = emb_table.T
    pooled_t = _pooled_sc(idx_t, emb_t)
    wt = W.T
    out_t = pl.pallas_call(
        _mm_body,
        grid=(NSTEPS,),
        in_specs=[
            pl.BlockSpec((EMB, BATCH), lambda i: (0, 0)),
            pl.BlockSpec((EMB, VB), lambda i: (0, i)),
        ],
        out_specs=pl.BlockSpec(memory_space=pl.ANY),
        out_shape=jax.ShapeDtypeStruct((VOCAB, BATCH), jnp.float32),
        scratch_shapes=[
            pltpu.VMEM((NBUF, VB, BATCH), jnp.float32),
            pltpu.SemaphoreType.DMA((NBUF,)),
        ],
        compiler_params=pltpu.CompilerParams(
            dimension_semantics=("arbitrary",),
        ),
    )(pooled_t, wt)
    return out_t.T


# speedup vs baseline: 1.0078x; 1.0011x over previous
import functools

import jax
import jax.numpy as jnp
from jax import lax
from jax.experimental import pallas as pl
from jax.experimental.pallas import tpu as pltpu
from jax.experimental.pallas import tpu_sc as plsc

VOCAB = 100000
EMB = 64
BATCH = 1024
CTX = 20

NUM_CORES = 2
NUM_SUBCORES = 16
NUM_WORKERS = NUM_CORES * NUM_SUBCORES
LANES = 16
VB = 2048
PLANES_PER_WORKER = EMB // NUM_WORKERS


def _pooled_sc(idx_t, emb_t):
    mesh = plsc.VectorSubcoreMesh(core_axis_name="c", subcore_axis_name="s")

    @functools.partial(
        pl.kernel,
        mesh=mesh,
        out_type=jax.ShapeDtypeStruct((EMB, BATCH), jnp.float32),
        scratch_types=[
            pltpu.VMEM((50080,), jnp.float32),
            pltpu.VMEM((50080,), jnp.float32),
            pltpu.VMEM((CTX, BATCH), jnp.int32),
            pltpu.VMEM((PLANES_PER_WORKER, BATCH), jnp.float32),
            pltpu.SemaphoreType.DMA((2,)),
        ],
        compiler_params=pltpu.CompilerParams(
            use_tc_tiling_on_sc=True, needs_layout_passes=False
        ),
    )
    def k(emb_hbm, idx_hbm, out_hbm, half_a, half_b, idx_v, pool_v, sems):
        wid = lax.axis_index("s") * NUM_CORES + lax.axis_index("c")
        halves = ((0, 49920), (49920, VOCAB - 49920))
        nchunks = PLANES_PER_WORKER * 2

        bufs = (half_a, half_b)

        def chunk_copy(ci, buf):
            r, h = divmod(ci, 2)
            off, sz = halves[h]
            kplane = wid * PLANES_PER_WORKER + r
            return pltpu.make_async_copy(
                emb_hbm.at[kplane].at[pl.ds(off, sz)],
                bufs[buf].at[pl.ds(0, sz)],
                sems.at[buf],
            )

        chunk_copy(0, 0).start()
        pltpu.sync_copy(idx_hbm, idx_v)
        for ci in range(nchunks):
            buf = ci % 2
            chunk_copy(ci, buf).wait()
            if ci + 1 < nchunks:
                chunk_copy(ci + 1, 1 - buf).start()
            r, h = divmod(ci, 2)
            off, sz = halves[h]
            lo = jnp.full((LANES,), off, jnp.int32)
            hi = jnp.full((LANES,), sz, jnp.int32)

            def group_body(g, carry, buf=buf, h=h, r=r, lo=lo, hi=hi):
                acc = jnp.zeros((LANES,), jnp.float32)
                for c in range(CTX):
                    idxs = idx_v[c, pl.ds(g * LANES, LANES)]
                    loc = idxs - lo
                    msk = jnp.logical_and(loc >= 0, loc < hi)
                    vals = plsc.load_gather(bufs[buf], [loc], mask=msk)
                    acc = acc + jnp.where(msk, vals, 0.0)
                if h == 0:
                    pool_v[r, pl.ds(g * LANES, LANES)] = acc
                else:
                    prev = pool_v[r, pl.ds(g * LANES, LANES)]
                    pool_v[r, pl.ds(g * LANES, LANES)] = prev + acc
                return carry

            lax.fori_loop(0, BATCH // LANES, group_body, 0)
        pltpu.sync_copy(
            pool_v, out_hbm.at[pl.ds(wid * PLANES_PER_WORKER, PLANES_PER_WORKER)]
        )

    return k(emb_t, idx_t)


NSTEPS = 49
TAIL = VOCAB - (NSTEPS - 1) * VB
NBUF = 4


def _mm_body(p_ref, wt_ref, o_hbm, acc, sems):
    i = pl.program_id(0)
    slot = lax.rem(i, NBUF)

    for k in range(NBUF):
        @pl.when(jnp.logical_and(i >= NBUF, slot == k))
        def _(k=k):
            pltpu.make_async_copy(
                acc.at[k],
                o_hbm.at[pl.ds((i - NBUF) * VB, VB)],
                sems.at[k],
            ).wait()

    acc[slot] = lax.dot_general(
        wt_ref[...],
        p_ref[...],
        dimension_numbers=(((0,), (0,)), ((), ())),
        preferred_element_type=jnp.float32,
    )

    for k in range(NBUF):
        @pl.when(jnp.logical_and(i < NSTEPS - 1, slot == k))
        def _(k=k):
            pltpu.make_async_copy(
                acc.at[k], o_hbm.at[pl.ds(i * VB, VB)], sems.at[k]
            ).start()

    @pl.when(i == NSTEPS - 1)
    def _():
        last = NSTEPS - 1
        pltpu.make_async_copy(
            acc.at[last % NBUF, pl.ds(0, TAIL)],
            o_hbm.at[pl.ds(last * VB, TAIL)],
            sems.at[last % NBUF],
        ).start()
        for j in range(NSTEPS - NBUF, NSTEPS - 1):
            pltpu.make_async_copy(
                acc.at[j % NBUF],
                o_hbm.at[pl.ds(j * VB, VB)],
                sems.at[j % NBUF],
            ).wait()
        pltpu.make_async_copy(
            acc.at[last % NBUF, pl.ds(0, TAIL)],
            o_hbm.at[pl.ds(last * VB, TAIL)],
            sems.at[last % NBUF],
        ).wait()


def kernel(x, emb_table, W):
    idx_t = x.T.astype(jnp.int32)
    emb_t = emb_table.T
    pooled_t = _pooled_sc(idx_t, emb_t)
    wt = W.T
    out_t = pl.pallas_call(
        _mm_body,
        grid=(NSTEPS,),
        in_specs=[
            pl.BlockSpec((EMB, BATCH), lambda i: (0, 0)),
            pl.BlockSpec((EMB, VB), lambda i: (0, i)),
        ],
        out_specs=pl.BlockSpec(memory_space=pl.ANY),
        out_shape=jax.ShapeDtypeStruct((VOCAB, BATCH), jnp.float32),
        scratch_shapes=[
            pltpu.VMEM((NBUF, VB, BATCH), jnp.float32),
            pltpu.SemaphoreType.DMA((NBUF,)),
        ],
        compiler_params=pltpu.CompilerParams(
            dimension_semantics=("arbitrary",),
        ),
    )(pooled_t, wt)
    return out_t.T
